# baseline (speedup 1.0000x reference)
import functools

import jax
import jax.numpy as jnp
from jax import lax
from jax.experimental import pallas as pl
from jax.experimental.pallas import tpu as pltpu
from jax.experimental.pallas import tpu_sc as plsc

BATCH = 4096
SEQ = 50
EMB_DIM = 64

NC = 2
NS = 16
NW = NC * NS

ROWS_PER_W = BATCH // NW
GRP = 8
N_GRP = ROWS_PER_W // GRP
NBUF = 2


def _make_kernel():
  mesh = plsc.VectorSubcoreMesh(
      core_axis_name="c", subcore_axis_name="s",
      num_cores=NC, num_subcores=NS)

  @functools.partial(
      pl.kernel,
      out_type=jax.ShapeDtypeStruct((BATCH, SEQ, EMB_DIM), jnp.float32),
      mesh=mesh,
      scratch_types=[
          pltpu.VMEM((ROWS_PER_W, SEQ), jnp.int32),
          pltpu.VMEM((NBUF, GRP, SEQ, EMB_DIM), jnp.float32),
          pltpu.SemaphoreType.DMA,
          pltpu.SemaphoreType.DMA,
      ],
      compiler_params=pltpu.CompilerParams(use_tc_tiling_on_sc=False),
  )
  def gather_kernel(idx_hbm, table_hbm, out_hbm, idx_v, rows_v, gsem, wsem):
    wid = lax.axis_index("s") * NC + lax.axis_index("c")
    brow0 = wid * ROWS_PER_W
    pltpu.sync_copy(idx_hbm.at[pl.ds(brow0, ROWS_PER_W)], idx_v)

    def fire_gathers(g, p):
      for r in range(GRP):
        pltpu.async_copy(
            table_hbm.at[idx_v.at[g * GRP + r]],
            rows_v.at[p, r], gsem)

    def wait_gathers(p):
      for r in range(GRP):
        pltpu.make_async_copy(
            table_hbm.at[idx_v.at[0]], rows_v.at[p, r], gsem
        ).wait()

    def fire_write(g, p):
      pltpu.async_copy(
          rows_v.at[p], out_hbm.at[pl.ds(brow0 + g * GRP, GRP)], wsem)

    def drain_write():
      pltpu.make_async_copy(
          rows_v.at[0], out_hbm.at[pl.ds(brow0, GRP)], wsem).wait()

    def body(g, carry):
      p = lax.rem(g, NBUF)
      fire_gathers(g, p)
      wait_gathers(p)

      @pl.when(g >= NBUF)
      def _():
        drain_write()

      fire_write(g, p)
      return carry

    lax.fori_loop(0, N_GRP, body, 0, unroll=False)
    for _ in range(NBUF):
      drain_write()

  return gather_kernel


_gather = _make_kernel()


@jax.jit
def kernel(inputs, embeddings):
  return _gather(inputs, embeddings)


# speedup vs baseline: 1.0122x; 1.0122x over previous
import functools

import jax
import jax.numpy as jnp
from jax import lax
from jax.experimental import pallas as pl
from jax.experimental.pallas import tpu as pltpu
from jax.experimental.pallas import tpu_sc as plsc

BATCH = 4096
SEQ = 50
EMB = 64
VOCAB = 1000000

NC = 2
NS = 16
NW = NC * NS

PAIR_BLK = 1024
NBLK = (VOCAB + 2 * PAIR_BLK - 1) // (2 * PAIR_BLK)
P_ROWS = NBLK * PAIR_BLK
LAST_IN_BLK = (VOCAB + PAIR_BLK - 1) // PAIR_BLK - 1


def _relayout_body(x1_ref, x2_ref, o_ref):
  o_ref[:, 0:EMB] = x1_ref[...].T
  o_ref[:, EMB:2 * EMB] = x2_ref[...].T


_relayout = pl.pallas_call(
    _relayout_body,
    grid=(NBLK,),
    in_specs=[
        pl.BlockSpec((EMB, PAIR_BLK), lambda i: (0, 2 * i)),
        pl.BlockSpec((EMB, PAIR_BLK),
                     lambda i: (0, jnp.minimum(2 * i + 1, LAST_IN_BLK))),
    ],
    out_specs=pl.BlockSpec((PAIR_BLK, 2 * EMB), lambda i: (i, 0)),
    out_shape=jax.ShapeDtypeStruct((P_ROWS, 2 * EMB), jnp.float32),
)


def _make_gather():
  mesh = plsc.VectorSubcoreMesh(
      core_axis_name="c", subcore_axis_name="s",
      num_cores=NC, num_subcores=NS)

  @functools.partial(
      pl.kernel,
      out_type=jax.ShapeDtypeStruct((SEQ, EMB, BATCH), jnp.float32),
      mesh=mesh,
      scratch_types=[
          pltpu.VMEM((SEQ * 128,), jnp.int32),
          pltpu.VMEM((SEQ * 128,), jnp.int32),
          pltpu.VMEM((SEQ * 128,), jnp.int32),
          pltpu.VMEM((2 * 128, 128), jnp.float32),
          pltpu.VMEM((2 * EMB, 128), jnp.float32),
          pltpu.SemaphoreType.DMA,
          pltpu.SemaphoreType.DMA,
      ],
      compiler_params=pltpu.CompilerParams(
          use_tc_tiling_on_sc=True, needs_layout_passes=False),
  )
  def gather_kernel(idx_hbm, tab_hbm, out_hbm,
                    idx_v, pv_v, co_v, rows_v, ot_v, gsem, wsem):
    w = lax.axis_index("s") * NC + lax.axis_index("c")
    b0 = w * 128

    for s in range(SEQ):
      pltpu.async_copy(idx_hbm.at[s, pl.ds(b0, 128)],
                       idx_v.at[pl.ds(s * 128, 128)], gsem)
    for s in range(SEQ):
      pltpu.make_async_copy(idx_hbm.at[0, pl.ds(b0, 128)],
                            idx_v.at[pl.ds(0, 128)], gsem).wait()

    def idx_math(u, carry):
      v = idx_v[pl.ds(u * 16, 16)]
      p = ((v >> 11) << 10) | (v & (PAIR_BLK - 1))
      co = ((v >> 10) & 1) << 6
      pv_v[pl.ds(u * 16, 16)] = p
      co_v[pl.ds(u * 16, 16)] = co
      return carry

    lax.fori_loop(0, SEQ * 8, idx_math, 0, unroll=False)

    def fire_gather(s, buf):
      pltpu.async_copy(
          tab_hbm.at[pv_v.at[pl.ds(s * 128, 128)]],
          rows_v.at[pl.ds(buf * 128, 128)], gsem)

    def wait_gather():
      pltpu.make_async_copy(
          tab_hbm.at[pv_v.at[pl.ds(0, 128)]],
          rows_v.at[pl.ds(0, 128)], gsem).wait()

    def fire_write(s, buf):
      pltpu.async_copy(
          ot_v.at[pl.ds(buf * EMB, EMB)],
          out_hbm.at[s, :, pl.ds(b0, 128)], wsem)

    def drain_write():
      pltpu.make_async_copy(
          ot_v.at[pl.ds(0, EMB)],
          out_hbm.at[0, :, pl.ds(b0, 128)], wsem).wait()

    lane = lax.iota(jnp.int32, 16)

    def unit(s, carry):
      buf = s & 1

      @pl.when(s + 1 < SEQ)
      def _():
        fire_gather(s + 1, buf ^ 1)

      wait_gather()

      @pl.when(s >= 2)
      def _():
        drain_write()

      rbase = buf * 128
      obase = buf * EMB

      def chunk(t, carry2):
        jvec = rbase + t * 16 + lane
        cov = co_v[pl.ds(s * 128 + t * 16, 16)]
        for c in range(EMB):
          vals = plsc.load_gather(rows_v, [jvec, cov + c])
          ot_v[obase + c, pl.ds(t * 16, 16)] = vals
        return carry2

      lax.fori_loop(0, 8, chunk, 0, unroll=False)
      fire_write(s, buf)
      return carry

    fire_gather(0, 0)
    lax.fori_loop(0, SEQ, unit, 0, unroll=False)
    drain_write()
    drain_write()

  return gather_kernel


_gather = _make_gather()


@jax.jit
def kernel(inputs, embeddings):
  tab2 = _relayout(embeddings.T, embeddings.T)
  out_t = _gather(inputs.T, tab2)
  return out_t.transpose(2, 0, 1)


# speedup vs baseline: 1.1983x; 1.1838x over previous
import functools

import jax
import jax.numpy as jnp
from jax import lax
from jax.experimental import pallas as pl
from jax.experimental.pallas import tpu as pltpu
from jax.experimental.pallas import tpu_sc as plsc

BATCH = 4096
SEQ = 50
EMB = 64
VOCAB = 1000000

NC = 2
NS = 16
NW = NC * NS

PAIR_BLK = 2048
PB_SH = 11
NBLK = (VOCAB + 2 * PAIR_BLK - 1) // (2 * PAIR_BLK)
P_ROWS = NBLK * PAIR_BLK
LAST_IN_BLK = (VOCAB + PAIR_BLK - 1) // PAIR_BLK - 1


def _relayout_body(x1_ref, x2_ref, o_ref):
  o_ref[:, 0:EMB] = x1_ref[...].T
  o_ref[:, EMB:2 * EMB] = x2_ref[...].T


_relayout = pl.pallas_call(
    _relayout_body,
    grid=(NBLK,),
    in_specs=[
        pl.BlockSpec((EMB, PAIR_BLK), lambda i: (0, 2 * i)),
        pl.BlockSpec((EMB, PAIR_BLK),
                     lambda i: (0, jnp.minimum(2 * i + 1, LAST_IN_BLK))),
    ],
    out_specs=pl.BlockSpec((PAIR_BLK, 2 * EMB), lambda i: (i, 0)),
    out_shape=jax.ShapeDtypeStruct((P_ROWS, 2 * EMB), jnp.float32),
)


def _make_gather():
  mesh = plsc.VectorSubcoreMesh(
      core_axis_name="c", subcore_axis_name="s",
      num_cores=NC, num_subcores=NS)

  @functools.partial(
      pl.kernel,
      out_type=jax.ShapeDtypeStruct((SEQ, EMB, BATCH), jnp.float32),
      mesh=mesh,
      scratch_types=[
          pltpu.VMEM((SEQ * 128,), jnp.int32),
          pltpu.VMEM((SEQ * 128,), jnp.int32),
          pltpu.VMEM((SEQ * 128,), jnp.int32),
          pltpu.VMEM((4 * 128, 128), jnp.float32),
          pltpu.VMEM((2 * EMB, 128), jnp.float32),
          pltpu.SemaphoreType.DMA,
          pltpu.SemaphoreType.DMA,
      ],
      compiler_params=pltpu.CompilerParams(
          use_tc_tiling_on_sc=True, needs_layout_passes=False),
  )
  def gather_kernel(idx_hbm, tab_hbm, out_hbm,
                    idx_v, pv_v, co_v, rows_v, ot_v, gsem, wsem):
    w = lax.axis_index("s") * NC + lax.axis_index("c")
    b0 = w * 128

    for s in range(SEQ):
      pltpu.async_copy(idx_hbm.at[s, pl.ds(b0, 128)],
                       idx_v.at[pl.ds(s * 128, 128)], gsem)
    for s in range(SEQ):
      pltpu.make_async_copy(idx_hbm.at[0, pl.ds(b0, 128)],
                            idx_v.at[pl.ds(0, 128)], gsem).wait()

    def idx_math(u, carry):
      v = idx_v[pl.ds(u * 16, 16)]
      p = ((v >> (PB_SH + 1)) << PB_SH) | (v & (PAIR_BLK - 1))
      co = ((v >> PB_SH) & 1) << 6
      pv_v[pl.ds(u * 16, 16)] = p
      co_v[pl.ds(u * 16, 16)] = co
      return carry

    lax.fori_loop(0, SEQ * 8, idx_math, 0, unroll=False)

    def fire_gather(s, buf):
      pltpu.async_copy(
          tab_hbm.at[pv_v.at[pl.ds(s * 128, 128)]],
          rows_v.at[pl.ds(buf * 128, 128)], gsem)

    def wait_gather():
      pltpu.make_async_copy(
          tab_hbm.at[pv_v.at[pl.ds(0, 128)]],
          rows_v.at[pl.ds(0, 128)], gsem).wait()

    def fire_write(s, buf):
      pltpu.async_copy(
          ot_v.at[pl.ds(buf * EMB, EMB)],
          out_hbm.at[s, :, pl.ds(b0, 128)], wsem)

    def drain_write():
      pltpu.make_async_copy(
          ot_v.at[pl.ds(0, EMB)],
          out_hbm.at[0, :, pl.ds(b0, 128)], wsem).wait()

    lane = lax.iota(jnp.int32, 16)

    def unit(s, carry):
      buf = s & 3
      obuf = s & 1

      @pl.when(s + 3 < SEQ)
      def _():
        fire_gather(s + 3, (s + 3) & 3)

      wait_gather()

      @pl.when(s >= 2)
      def _():
        drain_write()

      rbase = buf * 128
      obase = obuf * EMB

      def chunk(t, carry2):
        jvec = rbase + t * 16 + lane
        cov = co_v[pl.ds(s * 128 + t * 16, 16)]
        for c in range(EMB):
          vals = plsc.load_gather(rows_v, [jvec, cov + c])
          ot_v[obase + c, pl.ds(t * 16, 16)] = vals
        return carry2

      lax.fori_loop(0, 8, chunk, 0, unroll=False)
      fire_write(s, obuf)
      return carry

    fire_gather(0, 0)
    fire_gather(1, 1)
    fire_gather(2, 2)
    lax.fori_loop(0, SEQ, unit, 0, unroll=False)
    drain_write()
    drain_write()

  return gather_kernel


_gather = _make_gather()


@jax.jit
def kernel(inputs, embeddings):
  tab2 = _relayout(embeddings.T, embeddings.T)
  out_t = _gather(inputs.T, tab2)
  return out_t.transpose(2, 0, 1)


# speedup vs baseline: 1.5116x; 1.2615x over previous
import functools

import jax
import jax.numpy as jnp
from jax import lax
from jax.experimental import pallas as pl
from jax.experimental.pallas import tpu as pltpu
from jax.experimental.pallas import tpu_sc as plsc

BATCH = 4096
SEQ = 50
EMB = 64
VOCAB = 1000000

NC = 2
NS = 16
NW = NC * NS

PAIR_BLK = 2048
PB_SH = 11
NBLK = (VOCAB + 2 * PAIR_BLK - 1) // (2 * PAIR_BLK)
P_ROWS = NBLK * PAIR_BLK
LAST_IN_BLK = (VOCAB + PAIR_BLK - 1) // PAIR_BLK - 1


def _relayout_body(x1_ref, x2_ref, o_ref):
  o_ref[:, 0:EMB] = x1_ref[...].T
  o_ref[:, EMB:2 * EMB] = x2_ref[...].T


_relayout = pl.pallas_call(
    _relayout_body,
    grid=(NBLK,),
    in_specs=[
        pl.BlockSpec((EMB, PAIR_BLK), lambda i: (0, 2 * i)),
        pl.BlockSpec((EMB, PAIR_BLK),
                     lambda i: (0, jnp.minimum(2 * i + 1, LAST_IN_BLK))),
    ],
    out_specs=pl.BlockSpec((PAIR_BLK, 2 * EMB), lambda i: (i, 0)),
    out_shape=jax.ShapeDtypeStruct((P_ROWS, 2 * EMB), jnp.float32),
)


def _make_gather():
  mesh = plsc.VectorSubcoreMesh(
      core_axis_name="c", subcore_axis_name="s",
      num_cores=NC, num_subcores=NS)

  @functools.partial(
      pl.kernel,
      out_type=jax.ShapeDtypeStruct((SEQ, EMB, BATCH), jnp.float32),
      mesh=mesh,
      scratch_types=[
          pltpu.VMEM((SEQ * 128,), jnp.int32),
          pltpu.VMEM((SEQ * 128,), jnp.int32),
          pltpu.VMEM((SEQ * 128,), jnp.int32),
          pltpu.VMEM((2 * 128, 128), jnp.float32),
          pltpu.VMEM((2 * EMB, 128), jnp.float32),
          pltpu.SemaphoreType.DMA,
          pltpu.SemaphoreType.DMA,
      ],
      compiler_params=pltpu.CompilerParams(
          use_tc_tiling_on_sc=True, needs_layout_passes=False),
  )
  def gather_kernel(idx_hbm, tab_hbm, out_hbm,
                    idx_v, pv_v, co_v, rows_v, ot_v, gsem, wsem):
    w = lax.axis_index("s") * NC + lax.axis_index("c")
    b0 = w * 128

    for s in range(SEQ):
      pltpu.async_copy(idx_hbm.at[s, pl.ds(b0, 128)],
                       idx_v.at[pl.ds(s * 128, 128)], gsem)
    for s in range(SEQ):
      pltpu.make_async_copy(idx_hbm.at[0, pl.ds(b0, 128)],
                            idx_v.at[pl.ds(0, 128)], gsem).wait()

    def idx_math(u, carry):
      v = idx_v[pl.ds(u * 16, 16)]
      p = ((v >> (PB_SH + 1)) << PB_SH) | (v & (PAIR_BLK - 1))
      co = ((v >> PB_SH) & 1) << 6
      pv_v[pl.ds(u * 16, 16)] = p
      co_v[pl.ds(u * 16, 16)] = co
      return carry

    lax.fori_loop(0, SEQ * 8, idx_math, 0, unroll=False)

    def fire_gather(s, buf):
      pltpu.async_copy(
          tab_hbm.at[pv_v.at[pl.ds(s * 128, 128)]],
          rows_v.at[pl.ds(buf * 128, 128)], gsem)

    def wait_gather():
      pltpu.make_async_copy(
          tab_hbm.at[pv_v.at[pl.ds(0, 128)]],
          rows_v.at[pl.ds(0, 128)], gsem).wait()

    def fire_write(s, buf):
      pltpu.async_copy(
          ot_v.at[pl.ds(buf * EMB, EMB)],
          out_hbm.at[s, :, pl.ds(b0, 128)], wsem)

    def drain_write():
      pltpu.make_async_copy(
          ot_v.at[pl.ds(0, EMB)],
          out_hbm.at[0, :, pl.ds(b0, 128)], wsem).wait()

    lane = lax.iota(jnp.int32, 16)

    def assemble(s, par):
      DEPTH = 4
      for t in range(8):
        jvec = (par * 128 + t * 16) + lane
        cov = co_v[pl.ds(s * 128 + t * 16, 16)]
        pend = [plsc.load_gather(rows_v, [jvec, cov + c])
                for c in range(DEPTH)]
        for c in range(DEPTH, EMB):
          nxt = plsc.load_gather(rows_v, [jvec, cov + c])
          ot_v[par * EMB + (c - DEPTH), pl.ds(t * 16, 16)] = pend[0]
          pend = pend[1:] + [nxt]
        for k in range(DEPTH):
          ot_v[par * EMB + (EMB - DEPTH + k), pl.ds(t * 16, 16)] = pend[k]

    def unit(s, carry):
      par = s & 1
      wait_gather()

      @pl.when(s >= 2)
      def _():
        drain_write()

      @pl.when(par == 0)
      def _():
        assemble(s, 0)

      @pl.when(par == 1)
      def _():
        assemble(s, 1)

      @pl.when(s + 2 < SEQ)
      def _():
        fire_gather(s + 2, par)

      fire_write(s, par)
      return carry

    fire_gather(0, 0)
    fire_gather(1, 1)
    lax.fori_loop(0, SEQ, unit, 0, unroll=False)
    drain_write()
    drain_write()

  return gather_kernel


_gather = _make_gather()


@jax.jit
def kernel(inputs, embeddings):
  tab2 = _relayout(embeddings.T, embeddings.T)
  out_t = _gather(inputs.T, tab2)
  return out_t.transpose(2, 0, 1)


# speedup vs baseline: 1.7466x; 1.1555x over previous
import functools

import jax
import jax.numpy as jnp
from jax import lax
from jax.experimental import pallas as pl
from jax.experimental.pallas import tpu as pltpu
from jax.experimental.pallas import tpu_sc as plsc

BATCH = 4096
SEQ = 50
EMB = 64
VOCAB = 1000000

NC = 2
NS = 16
NW = NC * NS

PAIR_BLK = 4096
PB_SH = 12
NBLK = (VOCAB + 2 * PAIR_BLK - 1) // (2 * PAIR_BLK)
P_ROWS = NBLK * PAIR_BLK
LAST_IN_BLK = (VOCAB + PAIR_BLK - 1) // PAIR_BLK - 1


def _relayout_body(x1_ref, x2_ref, o_ref):
  o_ref[:, 0:EMB] = x1_ref[...].T
  o_ref[:, EMB:2 * EMB] = x2_ref[...].T


_relayout = pl.pallas_call(
    _relayout_body,
    grid=(NBLK,),
    in_specs=[
        pl.BlockSpec((EMB, PAIR_BLK), lambda i: (0, 2 * i)),
        pl.BlockSpec((EMB, PAIR_BLK),
                     lambda i: (0, jnp.minimum(2 * i + 1, LAST_IN_BLK))),
    ],
    out_specs=pl.BlockSpec((PAIR_BLK, 2 * EMB), lambda i: (i, 0)),
    out_shape=jax.ShapeDtypeStruct((P_ROWS, 2 * EMB), jnp.float32),
)


def _make_gather():
  mesh = plsc.VectorSubcoreMesh(
      core_axis_name="c", subcore_axis_name="s",
      num_cores=NC, num_subcores=NS)

  @functools.partial(
      pl.kernel,
      out_type=jax.ShapeDtypeStruct((SEQ, EMB, BATCH), jnp.float32),
      mesh=mesh,
      scratch_types=[
          pltpu.VMEM((SEQ * 128,), jnp.int32),
          pltpu.VMEM((SEQ * 128,), jnp.int32),
          pltpu.VMEM((SEQ * 128,), jnp.int32),
          pltpu.VMEM((2 * 128, 128), jnp.float32),
          pltpu.VMEM((2 * EMB, 128), jnp.float32),
          pltpu.SemaphoreType.DMA,
          pltpu.SemaphoreType.DMA,
      ],
      compiler_params=pltpu.CompilerParams(
          use_tc_tiling_on_sc=True, needs_layout_passes=False),
  )
  def gather_kernel(idx_hbm, tab_hbm, out_hbm,
                    idx_v, pv_v, co_v, rows_v, ot_v, gsem, wsem):
    w = lax.axis_index("s") * NC + lax.axis_index("c")
    b0 = w * 128

    for s in range(SEQ):
      pltpu.async_copy(idx_hbm.at[s, pl.ds(b0, 128)],
                       idx_v.at[pl.ds(s * 128, 128)], gsem)
    for s in range(SEQ):
      pltpu.make_async_copy(idx_hbm.at[0, pl.ds(b0, 128)],
                            idx_v.at[pl.ds(0, 128)], gsem).wait()

    def idx_math(u, carry):
      v = idx_v[pl.ds(u * 16, 16)]
      p = ((v >> (PB_SH + 1)) << PB_SH) | (v & (PAIR_BLK - 1))
      co = ((v >> PB_SH) & 1) << 6
      pv_v[pl.ds(u * 16, 16)] = p
      co_v[pl.ds(u * 16, 16)] = co
      return carry

    lax.fori_loop(0, SEQ * 8, idx_math, 0, unroll=False)

    def fire_gather(s, buf):
      pltpu.async_copy(
          tab_hbm.at[pv_v.at[pl.ds(s * 128, 128)]],
          rows_v.at[pl.ds(buf * 128, 128)], gsem)

    def wait_gather():
      pltpu.make_async_copy(
          tab_hbm.at[pv_v.at[pl.ds(0, 128)]],
          rows_v.at[pl.ds(0, 128)], gsem).wait()

    def fire_write(s, buf):
      pltpu.async_copy(
          ot_v.at[pl.ds(buf * EMB, EMB)],
          out_hbm.at[s, :, pl.ds(b0, 128)], wsem)

    def drain_write():
      pltpu.make_async_copy(
          ot_v.at[pl.ds(0, EMB)],
          out_hbm.at[0, :, pl.ds(b0, 128)], wsem).wait()

    lane = lax.iota(jnp.int32, 16)

    def assemble(s, par):
      DEPTH = 4
      for t in range(8):
        jvec = (par * 128 + t * 16) + lane
        cov = co_v[pl.ds(s * 128 + t * 16, 16)]
        pend = [plsc.load_gather(rows_v, [jvec, cov + c])
                for c in range(DEPTH)]
        for c in range(DEPTH, EMB):
          nxt = plsc.load_gather(rows_v, [jvec, cov + c])
          ot_v[par * EMB + (c - DEPTH), pl.ds(t * 16, 16)] = pend[0]
          pend = pend[1:] + [nxt]
        for k in range(DEPTH):
          ot_v[par * EMB + (EMB - DEPTH + k), pl.ds(t * 16, 16)] = pend[k]

    def unit(s, carry):
      par = s & 1
      wait_gather()

      @pl.when(s >= 2)
      def _():
        drain_write()

      @pl.when(par == 0)
      def _():
        assemble(s, 0)

      @pl.when(par == 1)
      def _():
        assemble(s, 1)

      @pl.when(s + 2 < SEQ)
      def _():
        fire_gather(s + 2, par)

      fire_write(s, par)
      return carry

    fire_gather(0, 0)
    fire_gather(1, 1)
    lax.fori_loop(0, SEQ, unit, 0, unroll=False)
    drain_write()
    drain_write()

  return gather_kernel


_gather = _make_gather()


@jax.jit
def kernel(inputs, embeddings):
  tab2 = _relayout(embeddings.T, embeddings.T)
  out_t = _gather(inputs.T, tab2)
  return out_t.transpose(2, 0, 1)


# speedup vs baseline: 1.8793x; 1.0760x over previous
import functools

import jax
import jax.numpy as jnp
from jax import lax
from jax.experimental import pallas as pl
from jax.experimental.pallas import tpu as pltpu
from jax.experimental.pallas import tpu_sc as plsc

BATCH = 4096
SEQ = 50
EMB = 64
VOCAB = 1000000

NC = 2
NS = 16
NW = NC * NS

PAIR_BLK = 8192
PB_SH = 13
NBLK = (VOCAB + 2 * PAIR_BLK - 1) // (2 * PAIR_BLK)
P_ROWS = NBLK * PAIR_BLK
LAST_IN_BLK = (VOCAB + PAIR_BLK - 1) // PAIR_BLK - 1


def _relayout_body(x1_ref, x2_ref, o_ref):
  o_ref[:, 0:EMB] = x1_ref[...].T
  o_ref[:, EMB:2 * EMB] = x2_ref[...].T


_relayout = pl.pallas_call(
    _relayout_body,
    grid=(NBLK,),
    in_specs=[
        pl.BlockSpec((EMB, PAIR_BLK), lambda i: (0, 2 * i)),
        pl.BlockSpec((EMB, PAIR_BLK),
                     lambda i: (0, jnp.minimum(2 * i + 1, LAST_IN_BLK))),
    ],
    out_specs=pl.BlockSpec((PAIR_BLK, 2 * EMB), lambda i: (i, 0)),
    out_shape=jax.ShapeDtypeStruct((P_ROWS, 2 * EMB), jnp.float32),
)


def _make_gather():
  mesh = plsc.VectorSubcoreMesh(
      core_axis_name="c", subcore_axis_name="s",
      num_cores=NC, num_subcores=NS)

  @functools.partial(
      pl.kernel,
      out_type=jax.ShapeDtypeStruct((SEQ, EMB, BATCH), jnp.float32),
      mesh=mesh,
      scratch_types=[
          pltpu.VMEM((SEQ * 128,), jnp.int32),
          pltpu.VMEM((SEQ * 128,), jnp.int32),
          pltpu.VMEM((SEQ * 128,), jnp.int32),
          pltpu.VMEM((3 * 128, 128), jnp.float32),
          pltpu.VMEM((3 * EMB, 128), jnp.float32),
          pltpu.SemaphoreType.DMA,
          pltpu.SemaphoreType.DMA,
      ],
      compiler_params=pltpu.CompilerParams(
          use_tc_tiling_on_sc=True, needs_layout_passes=False),
  )
  def gather_kernel(idx_hbm, tab_hbm, out_hbm,
                    idx_v, pv_v, co_v, rows_v, ot_v, gsem, wsem):
    w = lax.axis_index("s") * NC + lax.axis_index("c")
    b0 = w * 128

    for s in range(SEQ):
      pltpu.async_copy(idx_hbm.at[s, pl.ds(b0, 128)],
                       idx_v.at[pl.ds(s * 128, 128)], gsem)
    for s in range(SEQ):
      pltpu.make_async_copy(idx_hbm.at[0, pl.ds(b0, 128)],
                            idx_v.at[pl.ds(0, 128)], gsem).wait()

    def idx_math(u, carry):
      v = idx_v[pl.ds(u * 16, 16)]
      p = ((v >> (PB_SH + 1)) << PB_SH) | (v & (PAIR_BLK - 1))
      co = ((v >> PB_SH) & 1) << 6
      pv_v[pl.ds(u * 16, 16)] = p
      co_v[pl.ds(u * 16, 16)] = co
      return carry

    lax.fori_loop(0, SEQ * 8, idx_math, 0, unroll=False)

    def fire_gather(s, buf):
      pltpu.async_copy(
          tab_hbm.at[pv_v.at[pl.ds(s * 128, 128)]],
          rows_v.at[pl.ds(buf * 128, 128)], gsem)

    def wait_gather():
      pltpu.make_async_copy(
          tab_hbm.at[pv_v.at[pl.ds(0, 128)]],
          rows_v.at[pl.ds(0, 128)], gsem).wait()

    def fire_write(s, buf):
      pltpu.async_copy(
          ot_v.at[pl.ds(buf * EMB, EMB)],
          out_hbm.at[s, :, pl.ds(b0, 128)], wsem)

    def drain_write():
      pltpu.make_async_copy(
          ot_v.at[pl.ds(0, EMB)],
          out_hbm.at[0, :, pl.ds(b0, 128)], wsem).wait()

    lane = lax.iota(jnp.int32, 16)

    def assemble(s, par):
      DEPTH = 4
      for t in range(8):
        jvec = (par * 128 + t * 16) + lane
        cov = co_v[pl.ds(s * 128 + t * 16, 16)]
        pend = [plsc.load_gather(rows_v, [jvec, cov + c])
                for c in range(DEPTH)]
        for c in range(DEPTH, EMB):
          nxt = plsc.load_gather(rows_v, [jvec, cov + c])
          ot_v[par * EMB + (c - DEPTH), pl.ds(t * 16, 16)] = pend[0]
          pend = pend[1:] + [nxt]
        for k in range(DEPTH):
          ot_v[par * EMB + (EMB - DEPTH + k), pl.ds(t * 16, 16)] = pend[k]

    def unit(s, carry):
      par = lax.rem(s, 3)
      wait_gather()

      @pl.when(s + 2 < SEQ)
      def _():
        fire_gather(s + 2, lax.rem(s + 2, 3))

      @pl.when(s >= 3)
      def _():
        drain_write()

      for p in range(3):
        @pl.when(par == p)
        def _(p=p):
          assemble(s, p)

      fire_write(s, par)
      return carry

    fire_gather(0, 0)
    fire_gather(1, 1)
    lax.fori_loop(0, SEQ, unit, 0, unroll=False)
    drain_write()
    drain_write()
    drain_write()

  return gather_kernel


_gather = _make_gather()


@jax.jit
def kernel(inputs, embeddings):
  tab2 = _relayout(embeddings.T, embeddings.T)
  out_t = _gather(inputs.T, tab2)
  return out_t.transpose(2, 0, 1)


# speedup vs baseline: 1.9401x; 1.0323x over previous
import functools

import jax
import jax.numpy as jnp
from jax import lax
from jax.experimental import pallas as pl
from jax.experimental.pallas import tpu as pltpu
from jax.experimental.pallas import tpu_sc as plsc

BATCH = 4096
SEQ = 50
EMB = 64
VOCAB = 1000000

NC = 2
NS = 16
NW = NC * NS

PAIR_BLK = 16384
PB_SH = 14
NBLK = (VOCAB + 2 * PAIR_BLK - 1) // (2 * PAIR_BLK)
P_ROWS = NBLK * PAIR_BLK
LAST_IN_BLK = (VOCAB + PAIR_BLK - 1) // PAIR_BLK - 1


def _relayout_body(x1_ref, x2_ref, o_ref):
  o_ref[:, 0:EMB] = x1_ref[...].T
  o_ref[:, EMB:2 * EMB] = x2_ref[...].T


_relayout = pl.pallas_call(
    _relayout_body,
    grid=(NBLK,),
    in_specs=[
        pl.BlockSpec((EMB, PAIR_BLK), lambda i: (0, 2 * i)),
        pl.BlockSpec((EMB, PAIR_BLK),
                     lambda i: (0, jnp.minimum(2 * i + 1, LAST_IN_BLK))),
    ],
    out_specs=pl.BlockSpec((PAIR_BLK, 2 * EMB), lambda i: (i, 0)),
    out_shape=jax.ShapeDtypeStruct((P_ROWS, 2 * EMB), jnp.float32),
)


def _make_gather():
  mesh = plsc.VectorSubcoreMesh(
      core_axis_name="c", subcore_axis_name="s",
      num_cores=NC, num_subcores=NS)

  @functools.partial(
      pl.kernel,
      out_type=jax.ShapeDtypeStruct((SEQ, EMB, BATCH), jnp.float32),
      mesh=mesh,
      scratch_types=[
          pltpu.VMEM((SEQ * 128,), jnp.int32),
          pltpu.VMEM((SEQ * 128,), jnp.int32),
          pltpu.VMEM((SEQ * 128,), jnp.int32),
          pltpu.VMEM((3 * 128, 128), jnp.float32),
          pltpu.VMEM((3 * EMB, 128), jnp.float32),
          pltpu.SemaphoreType.DMA,
          pltpu.SemaphoreType.DMA,
      ],
      compiler_params=pltpu.CompilerParams(
          use_tc_tiling_on_sc=True, needs_layout_passes=False),
  )
  def gather_kernel(idx_hbm, tab_hbm, out_hbm,
                    idx_v, pv_v, co_v, rows_v, ot_v, gsem, wsem):
    w = lax.axis_index("s") * NC + lax.axis_index("c")
    b0 = w * 128

    for s in range(SEQ):
      pltpu.async_copy(idx_hbm.at[s, pl.ds(b0, 128)],
                       idx_v.at[pl.ds(s * 128, 128)], gsem)
    for s in range(SEQ):
      pltpu.make_async_copy(idx_hbm.at[0, pl.ds(b0, 128)],
                            idx_v.at[pl.ds(0, 128)], gsem).wait()

    def idx_math(u, carry):
      v = idx_v[pl.ds(u * 16, 16)]
      p = ((v >> (PB_SH + 1)) << PB_SH) | (v & (PAIR_BLK - 1))
      co = ((v >> PB_SH) & 1) << 6
      pv_v[pl.ds(u * 16, 16)] = p
      co_v[pl.ds(u * 16, 16)] = co
      return carry

    lax.fori_loop(0, SEQ * 8, idx_math, 0, unroll=False)

    def fire_gather(s, buf):
      for k in range(4):
        pltpu.async_copy(
            tab_hbm.at[pv_v.at[pl.ds(s * 128 + k * 32, 32)]],
            rows_v.at[pl.ds(buf * 128 + k * 32, 32)], gsem)

    def wait_gather():
      for _ in range(4):
        pltpu.make_async_copy(
            tab_hbm.at[pv_v.at[pl.ds(0, 32)]],
            rows_v.at[pl.ds(0, 32)], gsem).wait()

    def fire_write(s, buf):
      pltpu.async_copy(
          ot_v.at[pl.ds(buf * EMB, EMB)],
          out_hbm.at[s, :, pl.ds(b0, 128)], wsem)

    def drain_write():
      pltpu.make_async_copy(
          ot_v.at[pl.ds(0, EMB)],
          out_hbm.at[0, :, pl.ds(b0, 128)], wsem).wait()

    lane = lax.iota(jnp.int32, 16)

    def assemble(s, par):
      DEPTH = 4
      for t in range(8):
        jvec = (par * 128 + t * 16) + lane
        cov = co_v[pl.ds(s * 128 + t * 16, 16)]
        pend = [plsc.load_gather(rows_v, [jvec, cov + c])
                for c in range(DEPTH)]
        for c in range(DEPTH, EMB):
          nxt = plsc.load_gather(rows_v, [jvec, cov + c])
          ot_v[par * EMB + (c - DEPTH), pl.ds(t * 16, 16)] = pend[0]
          pend = pend[1:] + [nxt]
        for k in range(DEPTH):
          ot_v[par * EMB + (EMB - DEPTH + k), pl.ds(t * 16, 16)] = pend[k]

    def unit(s, carry):
      par = lax.rem(s, 3)
      wait_gather()

      @pl.when(s + 2 < SEQ)
      def _():
        fire_gather(s + 2, lax.rem(s + 2, 3))

      @pl.when(s >= 3)
      def _():
        drain_write()

      for p in range(3):
        @pl.when(par == p)
        def _(p=p):
          assemble(s, p)

      fire_write(s, par)
      return carry

    fire_gather(0, 0)
    fire_gather(1, 1)
    lax.fori_loop(0, SEQ, unit, 0, unroll=False)
    drain_write()
    drain_write()
    drain_write()

  return gather_kernel


_gather = _make_gather()


@jax.jit
def kernel(inputs, embeddings):
  tab2 = _relayout(embeddings.T, embeddings.T)
  out_t = _gather(inputs.T, tab2)
  return out_t.transpose(2, 0, 1)
